# initial kernel scaffold (unmeasured)
import jax
import jax.numpy as jnp
from jax import lax
from jax.experimental import pallas as pl
from jax.experimental.pallas import tpu as pltpu


def kernel(
    x,
):
    def body(*refs):
        pass

    out_shape = jax.ShapeDtypeStruct(..., jnp.float32)
    return pl.pallas_call(body, out_shape=out_shape)(...)


# baseline (device time: 100948 ns/iter reference)
import jax
import jax.numpy as jnp
from jax import lax
from jax.experimental import pallas as pl
from jax.experimental.pallas import tpu as pltpu


def kernel(x):
    x = x.reshape(x.shape[-2], x.shape[-1])
    m, n = x.shape

    def body(x_ref, out_ref, comm_ref, send_sems, recv_sems):
        my_x = lax.axis_index("x")
        my_y = lax.axis_index("y")
        y_nbr = (my_x, 1 - my_y)
        x_nbr = (1 - my_x, my_y)

        barrier_sem = pltpu.get_barrier_semaphore()
        for nbr in (y_nbr, x_nbr):
            pl.semaphore_signal(
                barrier_sem, inc=1,
                device_id=nbr, device_id_type=pl.DeviceIdType.MESH,
            )
        pl.semaphore_wait(barrier_sem, 2)

        rdma1 = pltpu.make_async_remote_copy(
            src_ref=x_ref,
            dst_ref=comm_ref.at[0],
            send_sem=send_sems.at[0],
            recv_sem=recv_sems.at[0],
            device_id=y_nbr,
            device_id_type=pl.DeviceIdType.MESH,
        )
        rdma1.start()
        rdma1.wait()
        out_ref[:, :] = x_ref[:, :] + comm_ref[0, :, :]

        rdma2 = pltpu.make_async_remote_copy(
            src_ref=out_ref,
            dst_ref=comm_ref.at[1],
            send_sem=send_sems.at[1],
            recv_sem=recv_sems.at[1],
            device_id=x_nbr,
            device_id_type=pl.DeviceIdType.MESH,
        )
        rdma2.start()
        rdma2.wait()
        out_ref[:, :] = out_ref[:, :] + comm_ref[1, :, :]

    return pl.pallas_call(
        body,
        out_shape=jax.ShapeDtypeStruct((m, n), jnp.float32),
        in_specs=[pl.BlockSpec(memory_space=pltpu.VMEM)],
        out_specs=pl.BlockSpec(memory_space=pltpu.VMEM),
        scratch_shapes=[
            pltpu.VMEM((2, m, n), jnp.float32),
            pltpu.SemaphoreType.DMA((2,)),
            pltpu.SemaphoreType.DMA((2,)),
        ],
        compiler_params=pltpu.CompilerParams(collective_id=0),
    )(x)


# device time: 46694 ns/iter; 2.1619x vs baseline; 2.1619x over previous
import jax
import jax.numpy as jnp
from jax import lax
from jax.experimental import pallas as pl
from jax.experimental.pallas import tpu as pltpu

MESH = pl.DeviceIdType.MESH


def kernel(x):
    x = x.reshape(x.shape[-2], x.shape[-1])
    m, n = x.shape
    q = m // 4

    def body(x_ref, out_ref, comm_ref, send_sems, recv_sems):
        my_x = lax.axis_index("x")
        my_y = lax.axis_index("y")
        y_nbr = (my_x, 1 - my_y)
        x_nbr = (1 - my_x, my_y)

        a_own = my_y * q
        a_oth = (1 - my_y) * q
        b_own = 2 * q + my_x * q
        b_oth = 2 * q + (1 - my_x) * q

        barrier_sem = pltpu.get_barrier_semaphore()
        for nbr in (y_nbr, x_nbr):
            pl.semaphore_signal(
                barrier_sem, inc=1, device_id=nbr, device_id_type=MESH,
            )
        pl.semaphore_wait(barrier_sem, 2)

        def exchange(slot_a, src_a, dst_a, slot_b, src_b, dst_b):
            ra = pltpu.make_async_remote_copy(
                src_ref=src_a, dst_ref=comm_ref.at[slot_a],
                send_sem=send_sems.at[slot_a], recv_sem=recv_sems.at[slot_a],
                device_id=dst_a, device_id_type=MESH,
            )
            rb = pltpu.make_async_remote_copy(
                src_ref=src_b, dst_ref=comm_ref.at[slot_b],
                send_sem=send_sems.at[slot_b], recv_sem=recv_sems.at[slot_b],
                device_id=dst_b, device_id_type=MESH,
            )
            ra.start()
            rb.start()
            ra.wait()
            rb.wait()

        exchange(
            0, x_ref.at[pl.ds(a_oth, q)], y_nbr,
            1, x_ref.at[pl.ds(b_oth, q)], x_nbr,
        )
        out_ref[pl.ds(a_own, q), :] = x_ref[pl.ds(a_own, q), :] + comm_ref[0]
        out_ref[pl.ds(b_own, q), :] = x_ref[pl.ds(b_own, q), :] + comm_ref[1]

        exchange(
            2, out_ref.at[pl.ds(a_own, q)], x_nbr,
            3, out_ref.at[pl.ds(b_own, q)], y_nbr,
        )
        out_ref[pl.ds(a_own, q), :] = out_ref[pl.ds(a_own, q), :] + comm_ref[2]
        out_ref[pl.ds(b_own, q), :] = out_ref[pl.ds(b_own, q), :] + comm_ref[3]

        exchange(
            4, out_ref.at[pl.ds(a_own, q)], y_nbr,
            5, out_ref.at[pl.ds(b_own, q)], x_nbr,
        )
        out_ref[pl.ds(a_oth, q), :] = comm_ref[4]
        out_ref[pl.ds(b_oth, q), :] = comm_ref[5]

    return pl.pallas_call(
        body,
        out_shape=jax.ShapeDtypeStruct((m, n), jnp.float32),
        in_specs=[pl.BlockSpec(memory_space=pltpu.VMEM)],
        out_specs=pl.BlockSpec(memory_space=pltpu.VMEM),
        scratch_shapes=[
            pltpu.VMEM((6, q, n), jnp.float32),
            pltpu.SemaphoreType.DMA((6,)),
            pltpu.SemaphoreType.DMA((6,)),
        ],
        compiler_params=pltpu.CompilerParams(collective_id=0),
    )(x)


# device time: 43129 ns/iter; 2.3406x vs baseline; 1.0827x over previous
import jax
import jax.numpy as jnp
from jax import lax
from jax.experimental import pallas as pl
from jax.experimental.pallas import tpu as pltpu

MESH = pl.DeviceIdType.MESH

C = 4


def kernel(x):
    x = x.reshape(x.shape[-2], x.shape[-1])
    m, n = x.shape
    q = m // 4
    ck = q // C

    def body(x_ref, out_ref, comm_ref, send_sems, recv_sems):
        my_x = lax.axis_index("x")
        my_y = lax.axis_index("y")
        y_nbr = (my_x, 1 - my_y)
        x_nbr = (1 - my_x, my_y)

        a_own = my_y * q
        a_oth = (1 - my_y) * q
        b_own = 2 * q + my_x * q
        b_oth = 2 * q + (1 - my_x) * q

        barrier_sem = pltpu.get_barrier_semaphore()
        for nbr in (y_nbr, x_nbr):
            pl.semaphore_signal(
                barrier_sem, inc=1, device_id=nbr, device_id_type=MESH,
            )
        pl.semaphore_wait(barrier_sem, 2)

        def mk(phase, half, c, src, nbr):
            buf = phase * 2 + half
            slot = buf * C + c
            return pltpu.make_async_remote_copy(
                src_ref=src,
                dst_ref=comm_ref.at[buf].at[pl.ds(c * ck, ck)],
                send_sem=send_sems.at[slot],
                recv_sem=recv_sems.at[slot],
                device_id=nbr,
                device_id_type=MESH,
            )

        rd = {}

        for c in range(C):
            rd[0, 0, c] = mk(0, 0, c, x_ref.at[pl.ds(a_oth + c * ck, ck)], y_nbr)
            rd[0, 1, c] = mk(0, 1, c, x_ref.at[pl.ds(b_oth + c * ck, ck)], x_nbr)
            rd[0, 0, c].start()
            rd[0, 1, c].start()

        for c in range(C):
            for half, own, nbr in ((0, a_own, x_nbr), (1, b_own, y_nbr)):
                rd[0, half, c].wait_recv()
                rows = pl.ds(own + c * ck, ck)
                out_ref[rows, :] = x_ref[rows, :] + comm_ref[half, pl.ds(c * ck, ck)]
                rd[1, half, c] = mk(1, half, c, out_ref.at[rows], nbr)
                rd[1, half, c].start()

        for c in range(C):
            for half, own, nbr in ((0, a_own, y_nbr), (1, b_own, x_nbr)):
                rd[1, half, c].wait_recv()
                rd[1, half, c].wait_send()
                rows = pl.ds(own + c * ck, ck)
                out_ref[rows, :] = (
                    out_ref[rows, :] + comm_ref[2 + half, pl.ds(c * ck, ck)]
                )
                rd[2, half, c] = mk(2, half, c, out_ref.at[rows], nbr)
                rd[2, half, c].start()

        for c in range(C):
            for half, oth in ((0, a_oth), (1, b_oth)):
                rd[2, half, c].wait_recv()
                out_ref[pl.ds(oth + c * ck, ck), :] = comm_ref[
                    4 + half, pl.ds(c * ck, ck)
                ]

        for c in range(C):
            for half in (0, 1):
                rd[0, half, c].wait_send()
                rd[2, half, c].wait_send()

    return pl.pallas_call(
        body,
        out_shape=jax.ShapeDtypeStruct((m, n), jnp.float32),
        in_specs=[pl.BlockSpec(memory_space=pltpu.VMEM)],
        out_specs=pl.BlockSpec(memory_space=pltpu.VMEM),
        scratch_shapes=[
            pltpu.VMEM((6, q, n), jnp.float32),
            pltpu.SemaphoreType.DMA((6 * C,)),
            pltpu.SemaphoreType.DMA((6 * C,)),
        ],
        compiler_params=pltpu.CompilerParams(collective_id=0),
    )(x)


# device time: 43100 ns/iter; 2.3422x vs baseline; 1.0007x over previous
import jax
import jax.numpy as jnp
from jax import lax
from jax.experimental import pallas as pl
from jax.experimental.pallas import tpu as pltpu

MESH = pl.DeviceIdType.MESH

C = 4


def kernel(x):
    x = x.reshape(x.shape[-2], x.shape[-1])
    m, n = x.shape
    q = m // 4
    ck = q // C

    def body(x_ref, out_ref, comm_ref, send_sems, recv_sems):
        my_x = lax.axis_index("x")
        my_y = lax.axis_index("y")
        y_nbr = (my_x, 1 - my_y)
        x_nbr = (1 - my_x, my_y)

        a_own = my_y * q
        a_oth = (1 - my_y) * q
        b_own = 2 * q + my_x * q
        b_oth = 2 * q + (1 - my_x) * q

        barrier_sem = pltpu.get_barrier_semaphore()
        for nbr in (y_nbr, x_nbr):
            pl.semaphore_signal(
                barrier_sem, inc=1, device_id=nbr, device_id_type=MESH,
            )
        pl.semaphore_wait(barrier_sem, 2)

        def mk(phase, half, c, src, nbr, dst=None):
            buf = phase * 2 + half
            slot = buf * C + c
            return pltpu.make_async_remote_copy(
                src_ref=src,
                dst_ref=comm_ref.at[buf].at[pl.ds(c * ck, ck)] if dst is None else dst,
                send_sem=send_sems.at[slot],
                recv_sem=recv_sems.at[slot],
                device_id=nbr,
                device_id_type=MESH,
            )

        rd = {}

        for c in range(C):
            rd[0, 0, c] = mk(0, 0, c, x_ref.at[pl.ds(a_oth + c * ck, ck)], y_nbr)
            rd[0, 1, c] = mk(0, 1, c, x_ref.at[pl.ds(b_oth + c * ck, ck)], x_nbr)
            rd[0, 0, c].start()
            rd[0, 1, c].start()

        for c in range(C):
            for half, own, nbr in ((0, a_own, x_nbr), (1, b_own, y_nbr)):
                rd[0, half, c].wait_recv()
                rows = pl.ds(own + c * ck, ck)
                out_ref[rows, :] = x_ref[rows, :] + comm_ref[half, pl.ds(c * ck, ck)]
                rd[1, half, c] = mk(1, half, c, out_ref.at[rows], nbr)
                rd[1, half, c].start()

        for c in range(C):
            for half, own, nbr in ((0, a_own, y_nbr), (1, b_own, x_nbr)):
                rd[1, half, c].wait_recv()
                rd[1, half, c].wait_send()
                rows = pl.ds(own + c * ck, ck)
                out_ref[rows, :] = (
                    out_ref[rows, :] + comm_ref[2 + half, pl.ds(c * ck, ck)]
                )
                rd[2, half, c] = mk(
                    2, half, c, out_ref.at[rows], nbr, dst=out_ref.at[rows]
                )
                rd[2, half, c].start()

        for c in range(C):
            for half in (0, 1):
                rd[2, half, c].wait_recv()

        for c in range(C):
            for half in (0, 1):
                rd[0, half, c].wait_send()
                rd[2, half, c].wait_send()

    return pl.pallas_call(
        body,
        out_shape=jax.ShapeDtypeStruct((m, n), jnp.float32),
        in_specs=[pl.BlockSpec(memory_space=pltpu.VMEM)],
        out_specs=pl.BlockSpec(memory_space=pltpu.VMEM),
        scratch_shapes=[
            pltpu.VMEM((4, q, n), jnp.float32),
            pltpu.SemaphoreType.DMA((6 * C,)),
            pltpu.SemaphoreType.DMA((6 * C,)),
        ],
        compiler_params=pltpu.CompilerParams(collective_id=0),
    )(x)
